# baseline (speedup 1.0000x reference)
import dataclasses
import functools

import jax
import jax.numpy as jnp
from jax import lax
from jax.experimental import pallas as pl
from jax.experimental.pallas import tpu as pltpu
from jax.experimental.pallas import tpu_sc as plsc

GRID = 0.01
NB = 65536
P1, P2, P3 = 73856093, 19349663, 83492791
B, N = 16, 50000
NCORE, NSUB, LANES = 2, 16, 16
PER_SUB = 3200
PTS_SUB = N // NSUB
CHUNKS = PER_SUB // LANES
JROWS = PER_SUB // 128
TW = 16
VJ = 13
VROWS = VJ * 128
ZROWS = 256
ROWS_PER_SUB = NB // NSUB
ENC_OUT = 1024
HID1 = 256
M_TILE = 2048
N_MT = NB // M_TILE


def _sc_body(b0, pts_hbm, zeros_hbm, out_hbm,
             pchunk, idxv, vals, zbuf, table):
    c = lax.axis_index("c")
    s = lax.axis_index("s")
    iota = lax.broadcasted_iota(jnp.int32, (LANES,), 0)
    col0 = jnp.zeros((LANES,), jnp.int32)
    col1 = col0 + 1
    col2 = col0 + 2
    col3 = col0 + 3
    gsz = jnp.float32(GRID)

    def vfloor(q):
        t = q.astype(jnp.int32)
        return jnp.where(t.astype(jnp.float32) > q, t - 1, t)

    pltpu.sync_copy(zeros_hbm, vals)
    pltpu.sync_copy(zeros_hbm.at[pl.ds(0, ZROWS)], zbuf)

    nb = out_hbm.shape[0]

    @pl.loop(0, nb // NCORE)
    def _batch(bi):
        b = c * (nb // NCORE) + bi
        bp = b0 + b

        for k in range(ROWS_PER_SUB // ZROWS):
            pltpu.sync_copy(zbuf, table.at[pl.ds(s * ROWS_PER_SUB + k * ZROWS, ZROWS)])
        plsc.subcore_barrier()

        pltpu.sync_copy(pts_hbm.at[bp, pl.ds(s * PTS_SUB, PTS_SUB)],
                        pchunk.at[pl.ds(0, PTS_SUB)])

        for off, nstream in ((0, VJ), (VJ * 128, JROWS - VJ)):
            @pl.loop(0, nstream * (128 // LANES))
            def _chunk(ch):
                r0 = ch * LANES
                g0 = off + r0
                prow = g0 + iota
                valid = prow < PTS_SUB
                px = jnp.where(valid, plsc.load_gather(pchunk, [prow, col0]), 0.0)
                py = jnp.where(valid, plsc.load_gather(pchunk, [prow, col1]), 0.0)
                pz = jnp.where(valid, plsc.load_gather(pchunk, [prow, col2]), 0.0)
                cx = vfloor(px / gsz)
                cy = vfloor(py / gsz)
                cz = vfloor(pz / gsz)
                h = ((cx * P1) ^ (cy * P2) ^ (cz * P3)) & (NB - 1)
                cnt = jnp.where(valid, jnp.float32(1.0), jnp.float32(0.0))
                rows = r0 + iota
                plsc.store_scatter(vals, [rows, col0], cnt)
                plsc.store_scatter(vals, [rows, col1], px)
                plsc.store_scatter(vals, [rows, col2], py)
                plsc.store_scatter(vals, [rows, col3], pz)
                jrow = jnp.full((LANES,), off // 128, jnp.int32) + (ch // 8)
                jcol = (ch & 7) * LANES + iota
                plsc.store_scatter(idxv, [jrow, jcol], h)

            @pl.loop(0, nstream)
            def _stream(j):
                pltpu.sync_copy(vals.at[pl.ds(j * 128, 128)],
                                table.at[idxv.at[off // 128 + j]], add=True)
        plsc.subcore_barrier()

        r = s * ROWS_PER_SUB
        pltpu.sync_copy(table.at[pl.ds(r, ROWS_PER_SUB)],
                        out_hbm.at[b, pl.ds(r, ROWS_PER_SUB)])
        plsc.subcore_barrier()


@functools.lru_cache(maxsize=2)
def _sc_call(nb, b0):
    mesh = plsc.VectorSubcoreMesh(core_axis_name="c", subcore_axis_name="s")
    cp = pltpu.CompilerParams()
    if "needs_layout_passes" in pltpu.CompilerParams.__dataclass_fields__:
        cp = dataclasses.replace(cp, needs_layout_passes=False)
    if "use_tc_tiling_on_sc" in pltpu.CompilerParams.__dataclass_fields__:
        cp = dataclasses.replace(cp, use_tc_tiling_on_sc=False)
    return pl.kernel(
        functools.partial(_sc_body, b0),
        mesh=mesh,
        compiler_params=cp,
        out_type=jax.ShapeDtypeStruct((nb, NB, TW), jnp.float32),
        scratch_types=[
            pltpu.VMEM((PER_SUB, 3), jnp.float32),
            pltpu.VMEM((JROWS, 128), jnp.int32),
            pltpu.VMEM((VROWS, TW), jnp.float32),
            pltpu.VMEM((ZROWS, TW), jnp.float32),
            pltpu.VMEM_SHARED((NB, TW), jnp.float32),
        ],
    )


def _enc_body(t_ref, w1_ref, b1_ref, w2_ref, b2_ref, enc_ref, acc_ref):
    m = pl.program_id(1)

    @pl.when(m == 0)
    def _():
        acc_ref[...] = jnp.full((1, ENC_OUT), -jnp.inf, jnp.float32)

    tt = t_ref[0]
    t = jnp.concatenate([tt[:, k * TW:(k + 1) * TW] for k in range(8)], axis=0)
    cnt = t[:, 0:1]
    colid = lax.broadcasted_iota(jnp.int32, (1, TW), 1)
    feat = jnp.where(colid == 0, t, t / jnp.maximum(cnt, 1.0))
    hdn = jnp.dot(feat.astype(jnp.bfloat16), w1_ref[...],
                  preferred_element_type=jnp.float32)
    hdn = jnp.maximum(hdn + b1_ref[...][None, :], 0.0)
    out = jnp.dot(hdn.astype(jnp.bfloat16), w2_ref[...],
                  preferred_element_type=jnp.float32)
    masked = jnp.where(cnt > 0.0, out, -jnp.inf)
    part = jnp.max(masked, axis=0)
    acc_ref[0, :] = jnp.maximum(acc_ref[0, :], part)

    @pl.when(m == N_MT - 1)
    def _():
        enc_ref[0, 0, :] = acc_ref[0, :] + b2_ref[...]


def _enc_call(tables, w1p, b1, w2bf, b2, interpret=False):
    nb = tables.shape[0]
    return pl.pallas_call(
        _enc_body,
        grid=(nb, N_MT),
        in_specs=[
            pl.BlockSpec((1, M_TILE // 8, 128), lambda b, m: (b, m, 0)),
            pl.BlockSpec((TW, HID1), lambda b, m: (0, 0)),
            pl.BlockSpec((HID1,), lambda b, m: (0,)),
            pl.BlockSpec((HID1, ENC_OUT), lambda b, m: (0, 0)),
            pl.BlockSpec((ENC_OUT,), lambda b, m: (0,)),
        ],
        out_specs=pl.BlockSpec((1, 1, ENC_OUT), lambda b, m: (b, 0, 0)),
        out_shape=jax.ShapeDtypeStruct((nb, 1, ENC_OUT), jnp.float32),
        scratch_shapes=[pltpu.VMEM((1, ENC_OUT), jnp.float32)],
        interpret=interpret,
    )(tables, w1p, b1, w2bf, b2).reshape(nb, ENC_OUT)


def _reg_body(enc_ref, wr1_ref, br1_ref, wr2_ref, br2_ref, out_ref):
    h = jnp.dot(enc_ref[...], wr1_ref[...], preferred_element_type=jnp.float32)
    h = jnp.maximum(h + br1_ref[...][None, :], 0.0)
    out = jnp.dot(h, wr2_ref[...], preferred_element_type=jnp.float32)
    out_ref[...] = out + br2_ref[...][None, :]


def _reg_call(enc, wr1, br1, wr2, br2, interpret=False):
    return pl.pallas_call(
        _reg_body,
        out_shape=jax.ShapeDtypeStruct((B, wr2.shape[1]), jnp.float32),
        interpret=interpret,
    )(enc, wr1, br1, wr2, br2)


def kernel(input, W1, b1, W2, b2, Wr1, br1, Wr2, br2):
    zeros_hbm = jnp.zeros((VROWS, TW), jnp.float32)
    w1p = jnp.pad(W1, ((0, TW - W1.shape[0]), (0, 0))).astype(jnp.bfloat16)
    w2bf = W2.astype(jnp.bfloat16)
    half = B // 2
    encs = []
    for h in range(2):
        tables = _sc_call(half, h * half)(input, zeros_hbm)
        tables = tables.reshape(half, NB // 8, 128)
        encs.append(_enc_call(tables, w1p, b1, w2bf, b2))
    enc = jnp.concatenate(encs, axis=0)
    pose = _reg_call(enc, Wr1, br1, Wr2, br2)
    return pose


# speedup vs baseline: 3.0519x; 3.0519x over previous
import dataclasses
import functools

import jax
import jax.numpy as jnp
from jax import lax
from jax.experimental import pallas as pl
from jax.experimental.pallas import tpu as pltpu
from jax.experimental.pallas import tpu_sc as plsc

GRID = 0.01
NB = 65536
P1, P2, P3 = 73856093, 19349663, 83492791
B, N = 16, 50000
NCORE, NSUB, LANES = 2, 16, 16
NPAD = 51200
PER_SUB = NPAD // NSUB
CHUNKS = PER_SUB // LANES
JROWS = PER_SUB // 128
TW = 16
VJ = 13
VROWS = VJ * 128
ZROWS = 256
ROWS_PER_SUB = NB // NSUB
ENC_OUT = 1024
HID1 = 256
M_TILE = 2048
N_MT = NB // M_TILE


def _sc_body(b0, pts_hbm, zeros_hbm, out_hbm,
             pchunk, idxv, vals, zbuf, table):
    c = lax.axis_index("c")
    s = lax.axis_index("s")
    iota = lax.broadcasted_iota(jnp.int32, (LANES,), 0)
    col0 = jnp.zeros((LANES,), jnp.int32)
    col1 = col0 + 1
    col2 = col0 + 2
    col3 = col0 + 3
    gsz = jnp.float32(GRID)

    def vfloor(q):
        t = q.astype(jnp.int32)
        return jnp.where(t.astype(jnp.float32) > q, t - 1, t)

    pltpu.sync_copy(zeros_hbm, vals)
    pltpu.sync_copy(zeros_hbm.at[pl.ds(0, ZROWS)], zbuf)

    nb = out_hbm.shape[0]

    @pl.loop(0, nb // NCORE)
    def _batch(bi):
        b = c * (nb // NCORE) + bi
        bp = b0 + b

        for k in range(ROWS_PER_SUB // ZROWS):
            pltpu.sync_copy(zbuf, table.at[pl.ds(s * ROWS_PER_SUB + k * ZROWS, ZROWS)])
        plsc.subcore_barrier()

        base = s * PER_SUB
        pltpu.sync_copy(pts_hbm.at[bp, pl.ds(base * 3, PER_SUB * 3)], pchunk)

        for off, nstream in ((0, VJ), (VJ * 128, JROWS - VJ)):
            @pl.loop(0, nstream * (128 // LANES))
            def _chunk(ch):
                r0 = ch * LANES
                g0 = off + r0
                fb = (g0 + iota) * 3
                px = plsc.load_gather(pchunk, [fb])
                py = plsc.load_gather(pchunk, [fb + 1])
                pz = plsc.load_gather(pchunk, [fb + 2])
                cx = vfloor(px / gsz)
                cy = vfloor(py / gsz)
                cz = vfloor(pz / gsz)
                h = ((cx * P1) ^ (cy * P2) ^ (cz * P3)) & (NB - 1)
                gidx = base + g0 + iota
                cnt = jnp.where(gidx < N, jnp.float32(1.0), jnp.float32(0.0))
                rows = r0 + iota
                plsc.store_scatter(vals, [rows, col0], cnt)
                plsc.store_scatter(vals, [rows, col1], px)
                plsc.store_scatter(vals, [rows, col2], py)
                plsc.store_scatter(vals, [rows, col3], pz)
                jrow = jnp.full((LANES,), off // 128, jnp.int32) + (ch // 8)
                jcol = (ch & 7) * LANES + iota
                plsc.store_scatter(idxv, [jrow, jcol], h)

            @pl.loop(0, nstream)
            def _stream(j):
                pltpu.sync_copy(vals.at[pl.ds(j * 128, 128)],
                                table.at[idxv.at[off // 128 + j]], add=True)
        plsc.subcore_barrier()

        r = s * ROWS_PER_SUB
        pltpu.sync_copy(table.at[pl.ds(r, ROWS_PER_SUB)],
                        out_hbm.at[b, pl.ds(r, ROWS_PER_SUB)])
        plsc.subcore_barrier()


@functools.lru_cache(maxsize=2)
def _sc_call(nb, b0):
    mesh = plsc.VectorSubcoreMesh(core_axis_name="c", subcore_axis_name="s")
    cp = pltpu.CompilerParams()
    if "needs_layout_passes" in pltpu.CompilerParams.__dataclass_fields__:
        cp = dataclasses.replace(cp, needs_layout_passes=False)
    if "use_tc_tiling_on_sc" in pltpu.CompilerParams.__dataclass_fields__:
        cp = dataclasses.replace(cp, use_tc_tiling_on_sc=False)
    return pl.kernel(
        functools.partial(_sc_body, b0),
        mesh=mesh,
        compiler_params=cp,
        out_type=jax.ShapeDtypeStruct((nb, NB, TW), jnp.float32),
        scratch_types=[
            pltpu.VMEM((PER_SUB * 3,), jnp.float32),
            pltpu.VMEM((JROWS, 128), jnp.int32),
            pltpu.VMEM((VROWS, TW), jnp.float32),
            pltpu.VMEM((ZROWS, TW), jnp.float32),
            pltpu.VMEM_SHARED((NB, TW), jnp.float32),
        ],
    )


def _enc_body(t_ref, w1_ref, b1_ref, w2_ref, b2_ref, enc_ref, acc_ref):
    m = pl.program_id(1)

    @pl.when(m == 0)
    def _():
        acc_ref[...] = jnp.full((1, ENC_OUT), -jnp.inf, jnp.float32)

    tt = t_ref[0]
    t = jnp.concatenate([tt[:, k * TW:(k + 1) * TW] for k in range(8)], axis=0)
    cnt = t[:, 0:1]
    colid = lax.broadcasted_iota(jnp.int32, (1, TW), 1)
    feat = jnp.where(colid == 0, t, t / jnp.maximum(cnt, 1.0))
    hdn = jnp.dot(feat.astype(jnp.bfloat16), w1_ref[...],
                  preferred_element_type=jnp.float32)
    hdn = jnp.maximum(hdn + b1_ref[...][None, :], 0.0)
    out = jnp.dot(hdn.astype(jnp.bfloat16), w2_ref[...],
                  preferred_element_type=jnp.float32)
    masked = jnp.where(cnt > 0.0, out, -jnp.inf)
    part = jnp.max(masked, axis=0)
    acc_ref[0, :] = jnp.maximum(acc_ref[0, :], part)

    @pl.when(m == N_MT - 1)
    def _():
        enc_ref[0, 0, :] = acc_ref[0, :] + b2_ref[...]


def _enc_call(tables, w1p, b1, w2bf, b2, interpret=False):
    nb = tables.shape[0]
    return pl.pallas_call(
        _enc_body,
        grid=(nb, N_MT),
        in_specs=[
            pl.BlockSpec((1, M_TILE // 8, 128), lambda b, m: (b, m, 0)),
            pl.BlockSpec((TW, HID1), lambda b, m: (0, 0)),
            pl.BlockSpec((HID1,), lambda b, m: (0,)),
            pl.BlockSpec((HID1, ENC_OUT), lambda b, m: (0, 0)),
            pl.BlockSpec((ENC_OUT,), lambda b, m: (0,)),
        ],
        out_specs=pl.BlockSpec((1, 1, ENC_OUT), lambda b, m: (b, 0, 0)),
        out_shape=jax.ShapeDtypeStruct((nb, 1, ENC_OUT), jnp.float32),
        scratch_shapes=[pltpu.VMEM((1, ENC_OUT), jnp.float32)],
        interpret=interpret,
    )(tables, w1p, b1, w2bf, b2).reshape(nb, ENC_OUT)


def _reg_body(enc_ref, wr1_ref, br1_ref, wr2_ref, br2_ref, out_ref):
    h = jnp.dot(enc_ref[...], wr1_ref[...], preferred_element_type=jnp.float32)
    h = jnp.maximum(h + br1_ref[...][None, :], 0.0)
    out = jnp.dot(h, wr2_ref[...], preferred_element_type=jnp.float32)
    out_ref[...] = out + br2_ref[...][None, :]


def _reg_call(enc, wr1, br1, wr2, br2, interpret=False):
    return pl.pallas_call(
        _reg_body,
        out_shape=jax.ShapeDtypeStruct((B, wr2.shape[1]), jnp.float32),
        interpret=interpret,
    )(enc, wr1, br1, wr2, br2)


def kernel(input, W1, b1, W2, b2, Wr1, br1, Wr2, br2):
    pts_flat = input.reshape(B, N * 3)
    pts_pad = jnp.pad(pts_flat, ((0, 0), (0, (NPAD - N) * 3)))
    zeros_hbm = jnp.zeros((VROWS, TW), jnp.float32)
    w1p = jnp.pad(W1, ((0, TW - W1.shape[0]), (0, 0))).astype(jnp.bfloat16)
    w2bf = W2.astype(jnp.bfloat16)
    half = B // 2
    encs = []
    for h in range(2):
        tables = _sc_call(half, h * half)(pts_pad, zeros_hbm)
        tables = tables.reshape(half, NB // 8, 128)
        encs.append(_enc_call(tables, w1p, b1, w2bf, b2))
    enc = jnp.concatenate(encs, axis=0)
    pose = _reg_call(enc, Wr1, br1, Wr2, br2)
    return pose


# speedup vs baseline: 3.2763x; 1.0735x over previous
import dataclasses
import functools

import jax
import jax.numpy as jnp
from jax import lax
from jax.experimental import pallas as pl
from jax.experimental.pallas import tpu as pltpu
from jax.experimental.pallas import tpu_sc as plsc

GRID = 0.01
NB = 65536
P1, P2, P3 = 73856093, 19349663, 83492791
B, N = 16, 50000
NCORE, NSUB, LANES = 2, 16, 16
NPAD = 51200
PER_SUB = NPAD // NSUB
CHUNKS = PER_SUB // LANES
JROWS = PER_SUB // 128
TW = 16
VJ = 13
VROWS = VJ * 128
ZROWS = 256
ROWS_PER_SUB = NB // NSUB
ENC_OUT = 1024
HID1 = 256
M_TILE = 4096
N_MT = NB // M_TILE


def _sc_body(b0, pts_hbm, zeros_hbm, out_hbm,
             pchunk, idxv, vals, zbuf, table):
    c = lax.axis_index("c")
    s = lax.axis_index("s")
    iota = lax.broadcasted_iota(jnp.int32, (LANES,), 0)
    col0 = jnp.zeros((LANES,), jnp.int32)
    col1 = col0 + 1
    col2 = col0 + 2
    col3 = col0 + 3
    gsz = jnp.float32(GRID)

    def vfloor(q):
        t = q.astype(jnp.int32)
        return jnp.where(t.astype(jnp.float32) > q, t - 1, t)

    pltpu.sync_copy(zeros_hbm, vals)
    pltpu.sync_copy(zeros_hbm.at[pl.ds(0, ZROWS)], zbuf)

    nb = out_hbm.shape[0]

    @pl.loop(0, nb // NCORE)
    def _batch(bi):
        b = c * (nb // NCORE) + bi
        bp = b0 + b

        for k in range(ROWS_PER_SUB // ZROWS):
            pltpu.sync_copy(zbuf, table.at[pl.ds(s * ROWS_PER_SUB + k * ZROWS, ZROWS)])
        plsc.subcore_barrier()

        base = s * PER_SUB
        pltpu.sync_copy(pts_hbm.at[bp, pl.ds(base * 3, PER_SUB * 3)], pchunk)

        for off, nstream in ((0, VJ), (VJ * 128, JROWS - VJ)):
            @pl.loop(0, nstream * (128 // LANES))
            def _chunk(ch):
                r0 = ch * LANES
                g0 = off + r0
                fb = (g0 + iota) * 3
                px = plsc.load_gather(pchunk, [fb])
                py = plsc.load_gather(pchunk, [fb + 1])
                pz = plsc.load_gather(pchunk, [fb + 2])
                cx = vfloor(px / gsz)
                cy = vfloor(py / gsz)
                cz = vfloor(pz / gsz)
                h = ((cx * P1) ^ (cy * P2) ^ (cz * P3)) & (NB - 1)
                gidx = base + g0 + iota
                cnt = jnp.where(gidx < N, jnp.float32(1.0), jnp.float32(0.0))
                rows = r0 + iota
                plsc.store_scatter(vals, [rows, col0], cnt)
                plsc.store_scatter(vals, [rows, col1], px)
                plsc.store_scatter(vals, [rows, col2], py)
                plsc.store_scatter(vals, [rows, col3], pz)
                jrow = jnp.full((LANES,), off // 128, jnp.int32) + (ch // 8)
                jcol = (ch & 7) * LANES + iota
                plsc.store_scatter(idxv, [jrow, jcol], h)

            @pl.loop(0, nstream)
            def _stream(j):
                pltpu.sync_copy(vals.at[pl.ds(j * 128, 128)],
                                table.at[idxv.at[off // 128 + j]], add=True)
        plsc.subcore_barrier()

        r = s * ROWS_PER_SUB
        pltpu.sync_copy(table.at[pl.ds(r, ROWS_PER_SUB)],
                        out_hbm.at[b, pl.ds(r, ROWS_PER_SUB)])
        plsc.subcore_barrier()


@functools.lru_cache(maxsize=2)
def _sc_call(nb, b0):
    mesh = plsc.VectorSubcoreMesh(core_axis_name="c", subcore_axis_name="s")
    cp = pltpu.CompilerParams()
    if "needs_layout_passes" in pltpu.CompilerParams.__dataclass_fields__:
        cp = dataclasses.replace(cp, needs_layout_passes=False)
    if "use_tc_tiling_on_sc" in pltpu.CompilerParams.__dataclass_fields__:
        cp = dataclasses.replace(cp, use_tc_tiling_on_sc=False)
    return pl.kernel(
        functools.partial(_sc_body, b0),
        mesh=mesh,
        compiler_params=cp,
        out_type=jax.ShapeDtypeStruct((nb, NB, TW), jnp.float32),
        scratch_types=[
            pltpu.VMEM((PER_SUB * 3,), jnp.float32),
            pltpu.VMEM((JROWS, 128), jnp.int32),
            pltpu.VMEM((VROWS, TW), jnp.float32),
            pltpu.VMEM((ZROWS, TW), jnp.float32),
            pltpu.VMEM_SHARED((NB, TW), jnp.float32),
        ],
    )


def _enc_body(t_ref, w1_ref, b1_ref, w2_ref, b2_ref, enc_ref, acc_ref):
    m = pl.program_id(1)

    @pl.when(m == 0)
    def _():
        acc_ref[...] = jnp.full((1, ENC_OUT), -jnp.inf, jnp.float32)

    tt = t_ref[0]
    t = jnp.concatenate([tt[:, k * TW:(k + 1) * TW] for k in range(8)], axis=0)
    cnt = t[:, 0:1]
    colid = lax.broadcasted_iota(jnp.int32, (1, TW), 1)
    feat = jnp.where(colid == 0, t, t / jnp.maximum(cnt, 1.0))
    hdn = jnp.dot(feat.astype(jnp.bfloat16), w1_ref[...],
                  preferred_element_type=jnp.float32)
    hdn = jnp.maximum(hdn + b1_ref[...][None, :], 0.0)
    out = jnp.dot(hdn.astype(jnp.bfloat16), w2_ref[...],
                  preferred_element_type=jnp.float32)
    masked = jnp.where(cnt > 0.0, out, -jnp.inf)
    part = jnp.max(masked, axis=0)
    acc_ref[0, :] = jnp.maximum(acc_ref[0, :], part)

    @pl.when(m == N_MT - 1)
    def _():
        enc_ref[0, 0, :] = acc_ref[0, :] + b2_ref[...]


def _enc_call(tables, w1p, b1, w2bf, b2, interpret=False):
    nb = tables.shape[0]
    return pl.pallas_call(
        _enc_body,
        grid=(nb, N_MT),
        in_specs=[
            pl.BlockSpec((1, M_TILE // 8, 128), lambda b, m: (b, m, 0)),
            pl.BlockSpec((TW, HID1), lambda b, m: (0, 0)),
            pl.BlockSpec((HID1,), lambda b, m: (0,)),
            pl.BlockSpec((HID1, ENC_OUT), lambda b, m: (0, 0)),
            pl.BlockSpec((ENC_OUT,), lambda b, m: (0,)),
        ],
        out_specs=pl.BlockSpec((1, 1, ENC_OUT), lambda b, m: (b, 0, 0)),
        out_shape=jax.ShapeDtypeStruct((nb, 1, ENC_OUT), jnp.float32),
        scratch_shapes=[pltpu.VMEM((1, ENC_OUT), jnp.float32)],
        interpret=interpret,
    )(tables, w1p, b1, w2bf, b2).reshape(nb, ENC_OUT)


def _reg_body(enc_ref, wr1_ref, br1_ref, wr2_ref, br2_ref, out_ref):
    h = jnp.dot(enc_ref[...], wr1_ref[...], preferred_element_type=jnp.float32)
    h = jnp.maximum(h + br1_ref[...][None, :], 0.0)
    out = jnp.dot(h, wr2_ref[...], preferred_element_type=jnp.float32)
    out_ref[...] = out + br2_ref[...][None, :]


def _reg_call(enc, wr1, br1, wr2, br2, interpret=False):
    return pl.pallas_call(
        _reg_body,
        out_shape=jax.ShapeDtypeStruct((B, wr2.shape[1]), jnp.float32),
        interpret=interpret,
    )(enc, wr1, br1, wr2, br2)


def kernel(input, W1, b1, W2, b2, Wr1, br1, Wr2, br2):
    pts_flat = input.reshape(B, N * 3)
    pts_pad = jnp.pad(pts_flat, ((0, 0), (0, (NPAD - N) * 3)))
    zeros_hbm = jnp.zeros((VROWS, TW), jnp.float32)
    w1p = jnp.pad(W1, ((0, TW - W1.shape[0]), (0, 0))).astype(jnp.bfloat16)
    w2bf = W2.astype(jnp.bfloat16)
    half = B // 2
    encs = []
    for h in range(2):
        tables = _sc_call(half, h * half)(pts_pad, zeros_hbm)
        tables = tables.reshape(half, NB // 8, 128)
        encs.append(_enc_call(tables, w1p, b1, w2bf, b2))
    enc = jnp.concatenate(encs, axis=0)
    pose = _reg_call(enc, Wr1, br1, Wr2, br2)
    return pose


# speedup vs baseline: 3.4034x; 1.0388x over previous
import dataclasses
import functools

import jax
import jax.numpy as jnp
from jax import lax
from jax.experimental import pallas as pl
from jax.experimental.pallas import tpu as pltpu
from jax.experimental.pallas import tpu_sc as plsc

GRID = 0.01
NB = 65536
P1, P2, P3 = 73856093, 19349663, 83492791
B, N = 16, 50000
NCORE, NSUB, LANES = 2, 16, 16
NPAD = 51200
PER_SUB = NPAD // NSUB
CHUNKS = PER_SUB // LANES
JROWS = PER_SUB // 128
TW = 16
VJ = 13
VROWS = VJ * 128
ZROWS = 256
ROWS_PER_SUB = NB // NSUB
ENC_OUT = 1024
HID1 = 256
M_TILE = 8192
N_MT = NB // M_TILE


def _sc_body(b0, pts_hbm, zeros_hbm, out_hbm,
             pchunk, idxv, vals, zbuf, table):
    c = lax.axis_index("c")
    s = lax.axis_index("s")
    iota = lax.broadcasted_iota(jnp.int32, (LANES,), 0)
    col0 = jnp.zeros((LANES,), jnp.int32)
    col1 = col0 + 1
    col2 = col0 + 2
    col3 = col0 + 3
    gsz = jnp.float32(GRID)

    def vfloor(q):
        t = q.astype(jnp.int32)
        return jnp.where(t.astype(jnp.float32) > q, t - 1, t)

    pltpu.sync_copy(zeros_hbm, vals)
    pltpu.sync_copy(zeros_hbm.at[pl.ds(0, ZROWS)], zbuf)

    nb = out_hbm.shape[0]

    @pl.loop(0, nb // NCORE)
    def _batch(bi):
        b = c * (nb // NCORE) + bi
        bp = b0 + b

        for k in range(ROWS_PER_SUB // ZROWS):
            pltpu.sync_copy(zbuf, table.at[pl.ds(s * ROWS_PER_SUB + k * ZROWS, ZROWS)])
        plsc.subcore_barrier()

        base = s * PER_SUB
        pltpu.sync_copy(pts_hbm.at[bp, pl.ds(base * 3, PER_SUB * 3)], pchunk)

        for off, nstream in ((0, VJ), (VJ * 128, JROWS - VJ)):
            @pl.loop(0, nstream * (128 // LANES))
            def _chunk(ch):
                r0 = ch * LANES
                g0 = off + r0
                fb = (g0 + iota) * 3
                px = plsc.load_gather(pchunk, [fb])
                py = plsc.load_gather(pchunk, [fb + 1])
                pz = plsc.load_gather(pchunk, [fb + 2])
                cx = vfloor(px / gsz)
                cy = vfloor(py / gsz)
                cz = vfloor(pz / gsz)
                h = ((cx * P1) ^ (cy * P2) ^ (cz * P3)) & (NB - 1)
                gidx = base + g0 + iota
                cnt = jnp.where(gidx < N, jnp.float32(1.0), jnp.float32(0.0))
                rows = r0 + iota
                plsc.store_scatter(vals, [rows, col0], cnt)
                plsc.store_scatter(vals, [rows, col1], px)
                plsc.store_scatter(vals, [rows, col2], py)
                plsc.store_scatter(vals, [rows, col3], pz)
                jrow = jnp.full((LANES,), off // 128, jnp.int32) + (ch // 8)
                jcol = (ch & 7) * LANES + iota
                plsc.store_scatter(idxv, [jrow, jcol], h)

            @pl.loop(0, nstream)
            def _stream(j):
                pltpu.sync_copy(vals.at[pl.ds(j * 128, 128)],
                                table.at[idxv.at[off // 128 + j]], add=True)
        plsc.subcore_barrier()

        r = s * ROWS_PER_SUB
        pltpu.sync_copy(table.at[pl.ds(r, ROWS_PER_SUB)],
                        out_hbm.at[b, pl.ds(r, ROWS_PER_SUB)])
        plsc.subcore_barrier()


@functools.lru_cache(maxsize=2)
def _sc_call(nb, b0):
    mesh = plsc.VectorSubcoreMesh(core_axis_name="c", subcore_axis_name="s")
    cp = pltpu.CompilerParams()
    if "needs_layout_passes" in pltpu.CompilerParams.__dataclass_fields__:
        cp = dataclasses.replace(cp, needs_layout_passes=False)
    if "use_tc_tiling_on_sc" in pltpu.CompilerParams.__dataclass_fields__:
        cp = dataclasses.replace(cp, use_tc_tiling_on_sc=False)
    return pl.kernel(
        functools.partial(_sc_body, b0),
        mesh=mesh,
        compiler_params=cp,
        out_type=jax.ShapeDtypeStruct((nb, NB, TW), jnp.float32),
        scratch_types=[
            pltpu.VMEM((PER_SUB * 3,), jnp.float32),
            pltpu.VMEM((JROWS, 128), jnp.int32),
            pltpu.VMEM((VROWS, TW), jnp.float32),
            pltpu.VMEM((ZROWS, TW), jnp.float32),
            pltpu.VMEM_SHARED((NB, TW), jnp.float32),
        ],
    )


def _enc_body(t_ref, w1_ref, b1_ref, w2_ref, b2_ref, enc_ref, acc_ref):
    m = pl.program_id(1)

    @pl.when(m == 0)
    def _():
        acc_ref[...] = jnp.full((1, ENC_OUT), -jnp.inf, jnp.float32)

    tt = t_ref[0]
    t = jnp.concatenate([tt[:, k * TW:(k + 1) * TW] for k in range(8)], axis=0)
    cnt = t[:, 0:1]
    colid = lax.broadcasted_iota(jnp.int32, (1, TW), 1)
    feat = jnp.where(colid == 0, t, t / jnp.maximum(cnt, 1.0))
    hdn = jnp.dot(feat.astype(jnp.bfloat16), w1_ref[...],
                  preferred_element_type=jnp.float32)
    hdn = jnp.maximum(hdn + b1_ref[...][None, :], 0.0)
    out = jnp.dot(hdn.astype(jnp.bfloat16), w2_ref[...],
                  preferred_element_type=jnp.float32)
    masked = jnp.where(cnt > 0.0, out, -jnp.inf)
    part = jnp.max(masked, axis=0)
    acc_ref[0, :] = jnp.maximum(acc_ref[0, :], part)

    @pl.when(m == N_MT - 1)
    def _():
        enc_ref[0, 0, :] = acc_ref[0, :] + b2_ref[...]


def _enc_call(tables, w1p, b1, w2bf, b2, interpret=False):
    nb = tables.shape[0]
    return pl.pallas_call(
        _enc_body,
        grid=(nb, N_MT),
        in_specs=[
            pl.BlockSpec((1, M_TILE // 8, 128), lambda b, m: (b, m, 0)),
            pl.BlockSpec((TW, HID1), lambda b, m: (0, 0)),
            pl.BlockSpec((HID1,), lambda b, m: (0,)),
            pl.BlockSpec((HID1, ENC_OUT), lambda b, m: (0, 0)),
            pl.BlockSpec((ENC_OUT,), lambda b, m: (0,)),
        ],
        out_specs=pl.BlockSpec((1, 1, ENC_OUT), lambda b, m: (b, 0, 0)),
        out_shape=jax.ShapeDtypeStruct((nb, 1, ENC_OUT), jnp.float32),
        scratch_shapes=[pltpu.VMEM((1, ENC_OUT), jnp.float32)],
        interpret=interpret,
    )(tables, w1p, b1, w2bf, b2).reshape(nb, ENC_OUT)


def _reg_body(enc_ref, wr1_ref, br1_ref, wr2_ref, br2_ref, out_ref):
    h = jnp.dot(enc_ref[...], wr1_ref[...], preferred_element_type=jnp.float32)
    h = jnp.maximum(h + br1_ref[...][None, :], 0.0)
    out = jnp.dot(h, wr2_ref[...], preferred_element_type=jnp.float32)
    out_ref[...] = out + br2_ref[...][None, :]


def _reg_call(enc, wr1, br1, wr2, br2, interpret=False):
    return pl.pallas_call(
        _reg_body,
        out_shape=jax.ShapeDtypeStruct((B, wr2.shape[1]), jnp.float32),
        interpret=interpret,
    )(enc, wr1, br1, wr2, br2)


def kernel(input, W1, b1, W2, b2, Wr1, br1, Wr2, br2):
    pts_flat = input.reshape(B, N * 3)
    pts_pad = jnp.pad(pts_flat, ((0, 0), (0, (NPAD - N) * 3)))
    zeros_hbm = jnp.zeros((VROWS, TW), jnp.float32)
    w1p = jnp.pad(W1, ((0, TW - W1.shape[0]), (0, 0))).astype(jnp.bfloat16)
    w2bf = W2.astype(jnp.bfloat16)
    half = B // 2
    encs = []
    for h in range(2):
        tables = _sc_call(half, h * half)(pts_pad, zeros_hbm)
        tables = tables.reshape(half, NB // 8, 128)
        encs.append(_enc_call(tables, w1p, b1, w2bf, b2))
    enc = jnp.concatenate(encs, axis=0)
    pose = _reg_call(enc, Wr1, br1, Wr2, br2)
    return pose
